# baseline (speedup 1.0000x reference)
import functools

import jax
import jax.numpy as jnp
from jax import lax
from jax.experimental import pallas as pl
from jax.experimental.pallas import tpu as pltpu
from jax.experimental.pallas import tpu_sc as plsc

_NUM_ENT = 100000
_NUM_REL = 1000
_D = 64
_B = 16384
_NC = 2
_NS = 16
_NW = _NC * _NS
_BPW = _B // _NW
_CH = 128
_NCH = _BPW // _CH
_L = 16

_ENT_BLK = 2000
_NSTEP = _NUM_ENT // _ENT_BLK


def _rsqrt_nr(x):
    i = plsc.bitcast(x, jnp.int32)
    i = jnp.int32(0x5F3759DF) - (i >> 1)
    y = plsc.bitcast(i, jnp.float32)
    for _ in range(3):
        y = y * (1.5 - 0.5 * x * y * y)
    return y


def _sc_body(h_hbm, r_hbm, t_hbm, n_hbm, ent_hbm, rel_hbm, norm_hbm,
             score_hbm, nscore_hbm,
             hi, ri, ti, ni, hbuf, tbuf, nbuf, rbuf, wbuf, s1, s2, sem):
    wid = lax.axis_index("s") * _NC + lax.axis_index("c")
    base = wid * _BPW
    pltpu.sync_copy(h_hbm.at[pl.ds(base, _BPW)], hi)
    pltpu.sync_copy(r_hbm.at[pl.ds(base, _BPW)], ri)
    pltpu.sync_copy(t_hbm.at[pl.ds(base, _BPW)], ti)
    pltpu.sync_copy(n_hbm.at[pl.ds(base, _BPW)], ni)

    for c in range(_NCH):
        cb = c * _CH
        cps = [
            pltpu.async_copy(ent_hbm.at[hi.at[pl.ds(cb, _CH)]], hbuf, sem),
            pltpu.async_copy(ent_hbm.at[ti.at[pl.ds(cb, _CH)]], tbuf, sem),
            pltpu.async_copy(ent_hbm.at[ni.at[pl.ds(cb, _CH)]], nbuf, sem),
            pltpu.async_copy(rel_hbm.at[ri.at[pl.ds(cb, _CH)]], rbuf, sem),
            pltpu.async_copy(norm_hbm.at[ri.at[pl.ds(cb, _CH)]], wbuf, sem),
        ]
        for cp in cps:
            cp.wait()

        for g in range(_CH // _L):
            rows = jnp.int32(g * _L) + lax.iota(jnp.int32, _L)

            def dots(d, carry, rows=rows):
                ww, wh, wt, wn = carry
                dd = jnp.broadcast_to(d, (_L,))
                wv = plsc.load_gather(wbuf, [rows, dd])
                hv = plsc.load_gather(hbuf, [rows, dd])
                tv = plsc.load_gather(tbuf, [rows, dd])
                nv = plsc.load_gather(nbuf, [rows, dd])
                return (ww + wv * wv, wh + wv * hv,
                        wt + wv * tv, wn + wv * nv)

            zero = jnp.zeros((_L,), jnp.float32)
            ww, wh, wt, wn = lax.fori_loop(
                0, _D, dots, (zero, zero, zero, zero), unroll=4)
            inv = 1.0 / jnp.maximum(ww, jnp.float32(1e-24))
            alpha = (wh - wt) * inv
            beta = (wh - wn) * inv

            def resid(d, carry, rows=rows, alpha=alpha, beta=beta):
                a1, a2 = carry
                dd = jnp.broadcast_to(d, (_L,))
                wv = plsc.load_gather(wbuf, [rows, dd])
                hv = plsc.load_gather(hbuf, [rows, dd])
                tv = plsc.load_gather(tbuf, [rows, dd])
                nv = plsc.load_gather(nbuf, [rows, dd])
                rv = plsc.load_gather(rbuf, [rows, dd])
                hr = hv + rv
                e1 = hr - tv - alpha * wv
                e2 = hr - nv - beta * wv
                return (a1 + e1 * e1, a2 + e2 * e2)

            a1, a2 = lax.fori_loop(0, _D, resid, (zero, zero), unroll=4)
            a1 = jnp.maximum(a1, jnp.float32(1e-30))
            a2 = jnp.maximum(a2, jnp.float32(1e-30))
            s1[pl.ds(cb + g * _L, _L)] = a1 * _rsqrt_nr(a1)
            s2[pl.ds(cb + g * _L, _L)] = a2 * _rsqrt_nr(a2)

    pltpu.sync_copy(s1, score_hbm.at[pl.ds(base, _BPW)])
    pltpu.sync_copy(s2, nscore_hbm.at[pl.ds(base, _BPW)])


def _tc_body(ent_ref, rel_ref, norm_ref, scale_ref, orth_ref):
    i = pl.program_id(0)
    x = ent_ref[...]
    nrm = jnp.sqrt(jnp.sum(x * x, axis=1, keepdims=True))
    part = jnp.sum(jnp.maximum(nrm - 1.0, 0.0))

    @pl.when(i == 0)
    def _():
        r = rel_ref[...]
        n = norm_ref[...]
        s = jnp.sum(r * n, axis=1, keepdims=True)
        rn = jnp.sqrt(jnp.sum(r * r, axis=1, keepdims=True))
        orth_ref[...] = s / rn

    prev = jnp.where(i == 0, jnp.zeros((1, 1), jnp.float32), scale_ref[...])
    tot = prev + jnp.broadcast_to(part, (1, 1))
    scale_ref[...] = jnp.where(i == _NSTEP - 1, tot / _NUM_ENT, tot)


def kernel(h, basic_r, t, neg_idx, ent_table, rel_table, norm_table):
    mesh = plsc.VectorSubcoreMesh(core_axis_name="c", subcore_axis_name="s")
    sc_call = pl.kernel(
        _sc_body,
        out_type=(
            jax.ShapeDtypeStruct((_B,), jnp.float32),
            jax.ShapeDtypeStruct((_B,), jnp.float32),
        ),
        mesh=mesh,
        compiler_params=pltpu.CompilerParams(
            needs_layout_passes=False, use_tc_tiling_on_sc=False),
        scratch_types=[
            pltpu.VMEM((_BPW,), jnp.int32),
            pltpu.VMEM((_BPW,), jnp.int32),
            pltpu.VMEM((_BPW,), jnp.int32),
            pltpu.VMEM((_BPW,), jnp.int32),
            pltpu.VMEM((_CH, _D), jnp.float32),
            pltpu.VMEM((_CH, _D), jnp.float32),
            pltpu.VMEM((_CH, _D), jnp.float32),
            pltpu.VMEM((_CH, _D), jnp.float32),
            pltpu.VMEM((_CH, _D), jnp.float32),
            pltpu.VMEM((_BPW,), jnp.float32),
            pltpu.VMEM((_BPW,), jnp.float32),
            pltpu.SemaphoreType.DMA,
        ],
    )
    score, neg_score = sc_call(h, basic_r, t, neg_idx,
                               ent_table, rel_table, norm_table)

    scale2d, orth2d = pl.pallas_call(
        _tc_body,
        grid=(_NSTEP,),
        in_specs=[
            pl.BlockSpec((_ENT_BLK, _D), lambda i: (i, 0)),
            pl.BlockSpec((_NUM_REL, _D), lambda i: (0, 0)),
            pl.BlockSpec((_NUM_REL, _D), lambda i: (0, 0)),
        ],
        out_specs=[
            pl.BlockSpec((1, 1), lambda i: (0, 0)),
            pl.BlockSpec((_NUM_REL, 1), lambda i: (0, 0)),
        ],
        out_shape=[
            jax.ShapeDtypeStruct((1, 1), jnp.float32),
            jax.ShapeDtypeStruct((_NUM_REL, 1), jnp.float32),
        ],
    )(ent_table, rel_table, norm_table)

    return (score, neg_score, scale2d[0, 0], orth2d[:, 0])


# speedup vs baseline: 1.7592x; 1.7592x over previous
import functools

import jax
import jax.numpy as jnp
from jax import lax
from jax.experimental import pallas as pl
from jax.experimental.pallas import tpu as pltpu
from jax.experimental.pallas import tpu_sc as plsc

_NUM_ENT = 100000
_NUM_REL = 1000
_D = 64
_B = 16384
_NC = 2
_NS = 16
_NW = _NC * _NS
_BPW = _B // _NW
_CH = 128
_NCH = _BPW // _CH
_L = 16

_ENT_BLK = 2000
_NSTEP = _NUM_ENT // _ENT_BLK


def _rsqrt_nr(x):
    i = plsc.bitcast(x, jnp.int32)
    i = jnp.int32(0x5F3759DF) - (i >> 1)
    y = plsc.bitcast(i, jnp.float32)
    for _ in range(3):
        y = y * (1.5 - 0.5 * x * y * y)
    return y


def _sc_body(h_hbm, r_hbm, t_hbm, n_hbm, ent_hbm, rel_hbm, norm_hbm,
             score_hbm, nscore_hbm,
             hi, ri, ti, ni, hbuf, tbuf, nbuf, rbuf, wbuf, s1, s2, sem):
    wid = lax.axis_index("s") * _NC + lax.axis_index("c")
    base = wid * _BPW
    pltpu.sync_copy(h_hbm.at[pl.ds(base, _BPW)], hi)
    pltpu.sync_copy(r_hbm.at[pl.ds(base, _BPW)], ri)
    pltpu.sync_copy(t_hbm.at[pl.ds(base, _BPW)], ti)
    pltpu.sync_copy(n_hbm.at[pl.ds(base, _BPW)], ni)

    for c in range(_NCH):
        cb = c * _CH
        cps = [
            pltpu.async_copy(ent_hbm.at[hi.at[pl.ds(cb, _CH)]], hbuf, sem),
            pltpu.async_copy(ent_hbm.at[ti.at[pl.ds(cb, _CH)]], tbuf, sem),
            pltpu.async_copy(ent_hbm.at[ni.at[pl.ds(cb, _CH)]], nbuf, sem),
            pltpu.async_copy(rel_hbm.at[ri.at[pl.ds(cb, _CH)]], rbuf, sem),
            pltpu.async_copy(norm_hbm.at[ri.at[pl.ds(cb, _CH)]], wbuf, sem),
        ]
        for cp in cps:
            cp.wait()

        lane0 = lax.iota(jnp.int32, _L) == 0

        def row(i, carry, cb=cb):
            w = [wbuf[i, pl.ds(_L * j, _L)] for j in range(4)]
            hh = [hbuf[i, pl.ds(_L * j, _L)] for j in range(4)]
            tt = [tbuf[i, pl.ds(_L * j, _L)] for j in range(4)]
            nn = [nbuf[i, pl.ds(_L * j, _L)] for j in range(4)]
            rr = [rbuf[i, pl.ds(_L * j, _L)] for j in range(4)]
            ww = w[0] * w[0] + w[1] * w[1] + w[2] * w[2] + w[3] * w[3]
            wh = w[0] * hh[0] + w[1] * hh[1] + w[2] * hh[2] + w[3] * hh[3]
            wt = w[0] * tt[0] + w[1] * tt[1] + w[2] * tt[2] + w[3] * tt[3]
            wn = w[0] * nn[0] + w[1] * nn[1] + w[2] * nn[2] + w[3] * nn[3]
            sww = jnp.broadcast_to(jnp.sum(ww), (_L,))
            swh = jnp.broadcast_to(jnp.sum(wh), (_L,))
            swt = jnp.broadcast_to(jnp.sum(wt), (_L,))
            swn = jnp.broadcast_to(jnp.sum(wn), (_L,))
            inv = 1.0 / jnp.maximum(sww, jnp.float32(1e-24))
            alpha = (swh - swt) * inv
            beta = (swh - swn) * inv
            acc1 = None
            acc2 = None
            for j in range(4):
                d = hh[j] + rr[j]
                e1 = d - tt[j] - alpha * w[j]
                e2 = d - nn[j] - beta * w[j]
                acc1 = e1 * e1 if acc1 is None else acc1 + e1 * e1
                acc2 = e2 * e2 if acc2 is None else acc2 + e2 * e2
            pos = jnp.broadcast_to(cb + i, (_L,))
            plsc.store_scatter(s1, [pos], jnp.broadcast_to(jnp.sum(acc1), (_L,)),
                               mask=lane0)
            plsc.store_scatter(s2, [pos], jnp.broadcast_to(jnp.sum(acc2), (_L,)),
                               mask=lane0)
            return carry

        lax.fori_loop(0, _CH, row, 0, unroll=4)

    for k in range(_BPW // _L):
        v1 = jnp.maximum(s1[pl.ds(_L * k, _L)], jnp.float32(1e-30))
        v2 = jnp.maximum(s2[pl.ds(_L * k, _L)], jnp.float32(1e-30))
        s1[pl.ds(_L * k, _L)] = v1 * _rsqrt_nr(v1)
        s2[pl.ds(_L * k, _L)] = v2 * _rsqrt_nr(v2)

    pltpu.sync_copy(s1, score_hbm.at[pl.ds(base, _BPW)])
    pltpu.sync_copy(s2, nscore_hbm.at[pl.ds(base, _BPW)])


def _tc_body(ent_ref, rel_ref, norm_ref, scale_ref, orth_ref):
    i = pl.program_id(0)
    x = ent_ref[...]
    nrm = jnp.sqrt(jnp.sum(x * x, axis=1, keepdims=True))
    part = jnp.sum(jnp.maximum(nrm - 1.0, 0.0))

    @pl.when(i == 0)
    def _():
        r = rel_ref[...]
        n = norm_ref[...]
        s = jnp.sum(r * n, axis=1, keepdims=True)
        rn = jnp.sqrt(jnp.sum(r * r, axis=1, keepdims=True))
        orth_ref[...] = s / rn

    prev = jnp.where(i == 0, jnp.zeros((1, 1), jnp.float32), scale_ref[...])
    tot = prev + jnp.broadcast_to(part, (1, 1))
    scale_ref[...] = jnp.where(i == _NSTEP - 1, tot / _NUM_ENT, tot)


def kernel(h, basic_r, t, neg_idx, ent_table, rel_table, norm_table):
    mesh = plsc.VectorSubcoreMesh(core_axis_name="c", subcore_axis_name="s")
    sc_call = pl.kernel(
        _sc_body,
        out_type=(
            jax.ShapeDtypeStruct((_B,), jnp.float32),
            jax.ShapeDtypeStruct((_B,), jnp.float32),
        ),
        mesh=mesh,
        compiler_params=pltpu.CompilerParams(
            needs_layout_passes=False, use_tc_tiling_on_sc=False),
        scratch_types=[
            pltpu.VMEM((_BPW,), jnp.int32),
            pltpu.VMEM((_BPW,), jnp.int32),
            pltpu.VMEM((_BPW,), jnp.int32),
            pltpu.VMEM((_BPW,), jnp.int32),
            pltpu.VMEM((_CH, _D), jnp.float32),
            pltpu.VMEM((_CH, _D), jnp.float32),
            pltpu.VMEM((_CH, _D), jnp.float32),
            pltpu.VMEM((_CH, _D), jnp.float32),
            pltpu.VMEM((_CH, _D), jnp.float32),
            pltpu.VMEM((_BPW,), jnp.float32),
            pltpu.VMEM((_BPW,), jnp.float32),
            pltpu.SemaphoreType.DMA,
        ],
    )
    score, neg_score = sc_call(h, basic_r, t, neg_idx,
                               ent_table, rel_table, norm_table)

    scale2d, orth2d = pl.pallas_call(
        _tc_body,
        grid=(_NSTEP,),
        in_specs=[
            pl.BlockSpec((_ENT_BLK, _D), lambda i: (i, 0)),
            pl.BlockSpec((_NUM_REL, _D), lambda i: (0, 0)),
            pl.BlockSpec((_NUM_REL, _D), lambda i: (0, 0)),
        ],
        out_specs=[
            pl.BlockSpec((1, 1), lambda i: (0, 0)),
            pl.BlockSpec((_NUM_REL, 1), lambda i: (0, 0)),
        ],
        out_shape=[
            jax.ShapeDtypeStruct((1, 1), jnp.float32),
            jax.ShapeDtypeStruct((_NUM_REL, 1), jnp.float32),
        ],
    )(ent_table, rel_table, norm_table)

    return (score, neg_score, scale2d[0, 0], orth2d[:, 0])


# speedup vs baseline: 1.9282x; 1.0961x over previous
import functools

import jax
import jax.numpy as jnp
from jax import lax
from jax.experimental import pallas as pl
from jax.experimental.pallas import tpu as pltpu
from jax.experimental.pallas import tpu_sc as plsc

_NUM_ENT = 100000
_NUM_REL = 1000
_D = 64
_B = 16384
_NC = 2
_NS = 16
_NW = _NC * _NS
_BPW = _B // _NW
_CH = 128
_NCH = _BPW // _CH
_L = 16

_ENT_BLK = 4000
_NSTEP = _NUM_ENT // _ENT_BLK


def _rsqrt_nr(x):
    i = plsc.bitcast(x, jnp.int32)
    i = jnp.int32(0x5F3759DF) - (i >> 1)
    y = plsc.bitcast(i, jnp.float32)
    for _ in range(3):
        y = y * (1.5 - 0.5 * x * y * y)
    return y


def _sc_body(h_hbm, r_hbm, t_hbm, n_hbm, ent_hbm, rel_hbm, norm_hbm,
             score_hbm, nscore_hbm,
             hi, ri, ti, ni, hbuf, tbuf, nbuf, rbuf, wbuf, s1, s2, sem):
    wid = lax.axis_index("s") * _NC + lax.axis_index("c")
    base = wid * _BPW
    pltpu.sync_copy(h_hbm.at[pl.ds(base, _BPW)], hi)
    pltpu.sync_copy(r_hbm.at[pl.ds(base, _BPW)], ri)
    pltpu.sync_copy(t_hbm.at[pl.ds(base, _BPW)], ti)
    pltpu.sync_copy(n_hbm.at[pl.ds(base, _BPW)], ni)

    for c in range(_NCH):
        cb = c * _CH
        cps = [
            pltpu.async_copy(ent_hbm.at[hi.at[pl.ds(cb, _CH)]], hbuf, sem),
            pltpu.async_copy(ent_hbm.at[ti.at[pl.ds(cb, _CH)]], tbuf, sem),
            pltpu.async_copy(ent_hbm.at[ni.at[pl.ds(cb, _CH)]], nbuf, sem),
            pltpu.async_copy(rel_hbm.at[ri.at[pl.ds(cb, _CH)]], rbuf, sem),
            pltpu.async_copy(norm_hbm.at[ri.at[pl.ds(cb, _CH)]], wbuf, sem),
        ]
        for cp in cps:
            cp.wait()

        lane0 = lax.iota(jnp.int32, _L) == 0

        def row(i, carry, cb=cb):
            w = [wbuf[i, pl.ds(_L * j, _L)] for j in range(4)]
            hh = [hbuf[i, pl.ds(_L * j, _L)] for j in range(4)]
            tt = [tbuf[i, pl.ds(_L * j, _L)] for j in range(4)]
            nn = [nbuf[i, pl.ds(_L * j, _L)] for j in range(4)]
            rr = [rbuf[i, pl.ds(_L * j, _L)] for j in range(4)]
            ww = w[0] * w[0] + w[1] * w[1] + w[2] * w[2] + w[3] * w[3]
            wh = w[0] * hh[0] + w[1] * hh[1] + w[2] * hh[2] + w[3] * hh[3]
            wt = w[0] * tt[0] + w[1] * tt[1] + w[2] * tt[2] + w[3] * tt[3]
            wn = w[0] * nn[0] + w[1] * nn[1] + w[2] * nn[2] + w[3] * nn[3]
            sww = jnp.broadcast_to(jnp.sum(ww), (_L,))
            swh = jnp.broadcast_to(jnp.sum(wh), (_L,))
            swt = jnp.broadcast_to(jnp.sum(wt), (_L,))
            swn = jnp.broadcast_to(jnp.sum(wn), (_L,))
            inv = 1.0 / jnp.maximum(sww, jnp.float32(1e-24))
            alpha = (swh - swt) * inv
            beta = (swh - swn) * inv
            acc1 = None
            acc2 = None
            for j in range(4):
                d = hh[j] + rr[j]
                e1 = d - tt[j] - alpha * w[j]
                e2 = d - nn[j] - beta * w[j]
                acc1 = e1 * e1 if acc1 is None else acc1 + e1 * e1
                acc2 = e2 * e2 if acc2 is None else acc2 + e2 * e2
            pos = jnp.broadcast_to(cb + i, (_L,))
            plsc.store_scatter(s1, [pos], jnp.broadcast_to(jnp.sum(acc1), (_L,)),
                               mask=lane0)
            plsc.store_scatter(s2, [pos], jnp.broadcast_to(jnp.sum(acc2), (_L,)),
                               mask=lane0)
            return carry

        lax.fori_loop(0, _CH, row, 0, unroll=4)

    for k in range(_BPW // _L):
        v1 = jnp.maximum(s1[pl.ds(_L * k, _L)], jnp.float32(1e-30))
        v2 = jnp.maximum(s2[pl.ds(_L * k, _L)], jnp.float32(1e-30))
        s1[pl.ds(_L * k, _L)] = v1 * _rsqrt_nr(v1)
        s2[pl.ds(_L * k, _L)] = v2 * _rsqrt_nr(v2)

    pltpu.sync_copy(s1, score_hbm.at[pl.ds(base, _BPW)])
    pltpu.sync_copy(s2, nscore_hbm.at[pl.ds(base, _BPW)])


def _tc_body(ent_ref, rel_ref, norm_ref, scale_ref, orth_ref):
    i = pl.program_id(0)
    x = ent_ref[...]
    s = jnp.sum(x * x, axis=1, keepdims=True)
    m = jnp.maximum(s, 1.0)
    part = jnp.sum(jnp.maximum(m * lax.rsqrt(m) - 1.0, 0.0))

    @pl.when(i == 0)
    def _():
        r = rel_ref[...]
        n = norm_ref[...]
        s = jnp.sum(r * n, axis=1, keepdims=True)
        rn = jnp.sqrt(jnp.sum(r * r, axis=1, keepdims=True))
        orth_ref[...] = s / rn

    prev = jnp.where(i == 0, jnp.zeros((1, 1), jnp.float32), scale_ref[...])
    tot = prev + jnp.broadcast_to(part, (1, 1))
    scale_ref[...] = jnp.where(i == _NSTEP - 1, tot / _NUM_ENT, tot)


def kernel(h, basic_r, t, neg_idx, ent_table, rel_table, norm_table):
    mesh = plsc.VectorSubcoreMesh(core_axis_name="c", subcore_axis_name="s")
    sc_call = pl.kernel(
        _sc_body,
        out_type=(
            jax.ShapeDtypeStruct((_B,), jnp.float32),
            jax.ShapeDtypeStruct((_B,), jnp.float32),
        ),
        mesh=mesh,
        compiler_params=pltpu.CompilerParams(
            needs_layout_passes=False, use_tc_tiling_on_sc=False),
        scratch_types=[
            pltpu.VMEM((_BPW,), jnp.int32),
            pltpu.VMEM((_BPW,), jnp.int32),
            pltpu.VMEM((_BPW,), jnp.int32),
            pltpu.VMEM((_BPW,), jnp.int32),
            pltpu.VMEM((_CH, _D), jnp.float32),
            pltpu.VMEM((_CH, _D), jnp.float32),
            pltpu.VMEM((_CH, _D), jnp.float32),
            pltpu.VMEM((_CH, _D), jnp.float32),
            pltpu.VMEM((_CH, _D), jnp.float32),
            pltpu.VMEM((_BPW,), jnp.float32),
            pltpu.VMEM((_BPW,), jnp.float32),
            pltpu.SemaphoreType.DMA,
        ],
    )
    score, neg_score = sc_call(h, basic_r, t, neg_idx,
                               ent_table, rel_table, norm_table)

    scale2d, orth2d = pl.pallas_call(
        _tc_body,
        grid=(_NSTEP,),
        in_specs=[
            pl.BlockSpec((_ENT_BLK, _D), lambda i: (i, 0)),
            pl.BlockSpec((_NUM_REL, _D), lambda i: (0, 0)),
            pl.BlockSpec((_NUM_REL, _D), lambda i: (0, 0)),
        ],
        out_specs=[
            pl.BlockSpec((1, 1), lambda i: (0, 0)),
            pl.BlockSpec((_NUM_REL, 1), lambda i: (0, 0)),
        ],
        out_shape=[
            jax.ShapeDtypeStruct((1, 1), jnp.float32),
            jax.ShapeDtypeStruct((_NUM_REL, 1), jnp.float32),
        ],
    )(ent_table, rel_table, norm_table)

    return (score, neg_score, scale2d[0, 0], orth2d[:, 0])
